# baseline (speedup 1.0000x reference)
import functools

import jax
import jax.numpy as jnp
import numpy as np
from jax import lax
from jax.experimental import pallas as pl
from jax.experimental.pallas import tpu as pltpu
from jax.experimental.pallas import tpu_sc as plsc

NC = 2
NS = 16
NW = NC * NS
L = 16

MAGIC = np.float32(1.5 * 2**23)
TABLE_N = 256
ROWS = 8
COLS = 1024
SPLIT = 4096 // COLS
NB = 8
LOOK = 4


CENTER = TABLE_N * 2 // 2
JBIAS = np.int32(0x4B400000 - 256)


def _build_sym_table(table_v, t2_v):
    @pl.loop(0, 2 * TABLE_N // L)
    def _b(i):
        k = i * L + lax.iota(jnp.int32, L)
        d = k - np.int32(CENTER)
        a = jnp.minimum(jnp.abs(d), np.int32(TABLE_N - 1))
        t2_v[pl.ds(i * L, L)] = plsc.load_gather(table_v, [a])


def _compute_slab(buf, t2_v):
    @plsc.parallel_loop(0, COLS // L)
    def _vec(i):
        s = pl.ds(i * L, L)
        for r in range(ROWS):
            xv = buf[r, s]
            u = xv * np.float32(2.0**6) + MAGIC
            j = plsc.bitcast(u, jnp.int32) - JBIAS
            idx = jnp.minimum(plsc.bitcast(j, jnp.uint32),
                              np.uint32(2 * TABLE_N - 1))
            tv = plsc.load_gather(t2_v, [plsc.bitcast(idx, jnp.int32)])
            relu = jnp.where(u >= MAGIC, xv, np.float32(0.0))
            buf[r, s] = relu - tv


def _secgelu_body(x_hbm, table_hbm, out_hbm, table_v, t2_v,
                  b0, b1, b2, b3, b4, b5, b6, b7, isem, osem, n_chunks):
    bufs = [b0, b1, b2, b3, b4, b5, b6, b7]
    wid = lax.axis_index("s") * NC + lax.axis_index("c")
    batch = wid // 16
    row0 = (wid % 16) * 512
    pltpu.sync_copy(table_hbm, table_v)
    _build_sym_table(table_v, t2_v)

    def slab(ref, jj, b):
        row = pl.multiple_of(row0 + (jj // SPLIT) * ROWS, ROWS)
        return ref.at[batch, pl.ds(row, ROWS),
                      pl.ds((b % SPLIT) * COLS, COLS)]

    for b in range(LOOK):
        pltpu.async_copy(slab(x_hbm, b, b), bufs[b], isem.at[b])

    @pl.loop(0, n_chunks, step=NB)
    def _ring(j):
        for b in range(NB):
            jj = j + b
            pltpu.make_async_copy(slab(x_hbm, jj, b), bufs[b],
                                  isem.at[b]).wait()
            _compute_slab(bufs[b], t2_v)
            pltpu.async_copy(bufs[b], slab(out_hbm, jj, b), osem.at[b])

            bp = (b + LOOK) % NB

            @pl.when(jj + LOOK < n_chunks)
            def _prefetch():
                @pl.when(jj >= NB - LOOK)
                def _reclaim():
                    pltpu.make_async_copy(
                        bufs[bp], slab(out_hbm, jj + LOOK - NB, bp),
                        osem.at[bp]).wait()

                pltpu.async_copy(slab(x_hbm, jj + LOOK, bp), bufs[bp],
                                 isem.at[bp])

    for b in range(NB):
        jj = n_chunks - NB + b
        pltpu.make_async_copy(bufs[b], slab(out_hbm, jj, b),
                              osem.at[b]).wait()


def kernel(x, table):
    assert x.shape == (2, 8192, 4096) and x.dtype == jnp.float32
    n_chunks = (8192 // 16) // ROWS * SPLIT
    mesh = plsc.VectorSubcoreMesh(
        core_axis_name="c", subcore_axis_name="s",
        num_cores=NC, num_subcores=NS)
    body = functools.partial(_secgelu_body, n_chunks=n_chunks)
    return pl.kernel(
        body,
        out_type=jax.ShapeDtypeStruct(x.shape, jnp.float32),
        mesh=mesh,
        scratch_types=[
            pltpu.VMEM((TABLE_N,), jnp.float32),
            pltpu.VMEM((2 * TABLE_N,), jnp.float32),
            *([pltpu.VMEM((ROWS, COLS), jnp.float32)] * NB),
            pltpu.SemaphoreType.DMA((NB,)),
            pltpu.SemaphoreType.DMA((NB,)),
        ],
        compiler_params=pltpu.CompilerParams(needs_layout_passes=False),
        name="secgelu_sc",
    )(x, table)


# speedup vs baseline: 1.0382x; 1.0382x over previous
import functools

import jax
import jax.numpy as jnp
import numpy as np
from jax import lax
from jax.experimental import pallas as pl
from jax.experimental.pallas import tpu as pltpu
from jax.experimental.pallas import tpu_sc as plsc

NC = 2
NS = 16
NW = NC * NS
L = 16

MAGIC = np.float32(1.5 * 2**23)
TABLE_N = 256
ROWS = 8
COLS = 2048
SPLIT = 4096 // COLS
NB = 4
LOOK = 2


CENTER = TABLE_N * 2 // 2
JBIAS = np.int32(0x4B400000 - 256)


def _build_sym_table(table_v, t2_v):
    @pl.loop(0, 2 * TABLE_N // L)
    def _b(i):
        k = i * L + lax.iota(jnp.int32, L)
        d = k - np.int32(CENTER)
        a = jnp.minimum(jnp.abs(d), np.int32(TABLE_N - 1))
        t2_v[pl.ds(i * L, L)] = plsc.load_gather(table_v, [a])


def _compute_slab(buf, t2_v):
    @plsc.parallel_loop(0, COLS // L)
    def _vec(i):
        s = pl.ds(i * L, L)
        for r in range(ROWS):
            xv = buf[r, s]
            u = xv * np.float32(2.0**6) + MAGIC
            j = plsc.bitcast(u, jnp.int32) - JBIAS
            idx = jnp.minimum(plsc.bitcast(j, jnp.uint32),
                              np.uint32(2 * TABLE_N - 1))
            tv = plsc.load_gather(t2_v, [plsc.bitcast(idx, jnp.int32)])
            relu = jnp.where(u >= MAGIC, xv, np.float32(0.0))
            buf[r, s] = relu - tv


def _secgelu_body(x_hbm, table_hbm, out_hbm, table_v, t2_v,
                  b0, b1, b2, b3, isem, osem, n_chunks):
    bufs = [b0, b1, b2, b3]
    wid = lax.axis_index("s") * NC + lax.axis_index("c")
    batch = wid // 16
    row0 = (wid % 16) * 512

    def slab(ref, jj, b):
        row = pl.multiple_of(row0 + (jj // SPLIT) * ROWS, ROWS)
        return ref.at[batch, pl.ds(row, ROWS),
                      pl.ds((b % SPLIT) * COLS, COLS)]

    for b in range(LOOK):
        pltpu.async_copy(slab(x_hbm, b, b), bufs[b], isem.at[b])
    pltpu.sync_copy(table_hbm, table_v)
    _build_sym_table(table_v, t2_v)

    @pl.loop(0, n_chunks, step=NB)
    def _ring(j):
        for b in range(NB):
            jj = j + b
            pltpu.make_async_copy(slab(x_hbm, jj, b), bufs[b],
                                  isem.at[b]).wait()
            _compute_slab(bufs[b], t2_v)
            pltpu.async_copy(bufs[b], slab(out_hbm, jj, b), osem.at[b])

            bp = (b + LOOK) % NB

            @pl.when(jj + LOOK < n_chunks)
            def _prefetch():
                @pl.when(jj >= NB - LOOK)
                def _reclaim():
                    pltpu.make_async_copy(
                        bufs[bp], slab(out_hbm, jj + LOOK - NB, bp),
                        osem.at[bp]).wait()

                pltpu.async_copy(slab(x_hbm, jj + LOOK, bp), bufs[bp],
                                 isem.at[bp])

    for b in range(NB):
        jj = n_chunks - NB + b
        pltpu.make_async_copy(bufs[b], slab(out_hbm, jj, b),
                              osem.at[b]).wait()


def kernel(x, table):
    assert x.shape == (2, 8192, 4096) and x.dtype == jnp.float32
    n_chunks = (8192 // 16) // ROWS * SPLIT
    mesh = plsc.VectorSubcoreMesh(
        core_axis_name="c", subcore_axis_name="s",
        num_cores=NC, num_subcores=NS)
    body = functools.partial(_secgelu_body, n_chunks=n_chunks)
    return pl.kernel(
        body,
        out_type=jax.ShapeDtypeStruct(x.shape, jnp.float32),
        mesh=mesh,
        scratch_types=[
            pltpu.VMEM((TABLE_N,), jnp.float32),
            pltpu.VMEM((2 * TABLE_N,), jnp.float32),
            pltpu.VMEM((ROWS, COLS), jnp.float32),
            pltpu.VMEM((ROWS, COLS), jnp.float32),
            pltpu.VMEM((ROWS, COLS), jnp.float32),
            pltpu.VMEM((ROWS, COLS), jnp.float32),
            pltpu.SemaphoreType.DMA((NB,)),
            pltpu.SemaphoreType.DMA((NB,)),
        ],
        compiler_params=pltpu.CompilerParams(needs_layout_passes=False),
        name="secgelu_sc",
    )(x, table)


# speedup vs baseline: 1.0591x; 1.0202x over previous
import functools

import jax
import jax.numpy as jnp
import numpy as np
from jax import lax
from jax.experimental import pallas as pl
from jax.experimental.pallas import tpu as pltpu
from jax.experimental.pallas import tpu_sc as plsc

NC = 2
NS = 16
L = 16

MAGIC = np.float32(1.5 * 2**23)
TABLE_N = 256
ROWS = 8
COLS = 2048
SPLIT = 4096 // COLS
NB = 4
LOOK = 3


CENTER = TABLE_N
JBIAS = np.int32(0x4B400000 - 256)


def _build_sym_table(table_v, t2_v):
    @pl.loop(0, 2 * TABLE_N // L)
    def _b(i):
        k = i * L + lax.iota(jnp.int32, L)
        d = k - np.int32(CENTER)
        a = jnp.minimum(jnp.abs(d), np.int32(TABLE_N - 1))
        t2_v[pl.ds(i * L, L)] = plsc.load_gather(table_v, [a])


def _compute_slab(buf, t2_v):
    @plsc.parallel_loop(0, COLS // L)
    def _vec(i):
        s = pl.ds(i * L, L)
        for r in range(ROWS):
            xv = buf[r, s]
            u = xv * np.float32(2.0**6) + MAGIC
            j = plsc.bitcast(u, jnp.int32) - JBIAS
            idx = jnp.minimum(plsc.bitcast(j, jnp.uint32),
                              np.uint32(2 * TABLE_N - 1))
            tv = plsc.load_gather(t2_v, [plsc.bitcast(idx, jnp.int32)])
            relu = jnp.where(u >= MAGIC, xv, np.float32(0.0))
            buf[r, s] = relu - tv


def _secgelu_body(x_hbm, table_hbm, out_hbm, table_v, t2_v,
                  b0, b1, b2, b3, isem, osem, n_chunks):
    bufs = [b0, b1, b2, b3]
    wid = lax.axis_index("s") * NC + lax.axis_index("c")
    batch = wid // 16
    row0 = (wid % 16) * 512

    def slab(ref, jj, b):
        row = pl.multiple_of(row0 + (jj // SPLIT) * ROWS, ROWS)
        return ref.at[batch, pl.ds(row, ROWS),
                      pl.ds((b % SPLIT) * COLS, COLS)]

    for b in range(LOOK):
        pltpu.async_copy(slab(x_hbm, b, b), bufs[b], isem.at[b])
    pltpu.sync_copy(table_hbm, table_v)
    _build_sym_table(table_v, t2_v)

    @pl.loop(0, n_chunks, step=NB)
    def _ring(j):
        for b in range(NB):
            jj = j + b
            pltpu.make_async_copy(slab(x_hbm, jj, b), bufs[b],
                                  isem.at[b]).wait()
            _compute_slab(bufs[b], t2_v)
            pltpu.async_copy(bufs[b], slab(out_hbm, jj, b), osem.at[b])

            bp = (b + LOOK) % NB

            @pl.when(jj + LOOK < n_chunks)
            def _prefetch():
                @pl.when(jj >= NB - LOOK)
                def _reclaim():
                    pltpu.make_async_copy(
                        bufs[bp], slab(out_hbm, jj + LOOK - NB, bp),
                        osem.at[bp]).wait()

                pltpu.async_copy(slab(x_hbm, jj + LOOK, bp), bufs[bp],
                                 isem.at[bp])

    for b in range(NB):
        jj = n_chunks - NB + b
        pltpu.make_async_copy(bufs[b], slab(out_hbm, jj, b),
                              osem.at[b]).wait()


def kernel(x, table):
    assert x.shape == (2, 8192, 4096) and x.dtype == jnp.float32
    n_chunks = (8192 // 16) // ROWS * SPLIT
    mesh = plsc.VectorSubcoreMesh(
        core_axis_name="c", subcore_axis_name="s",
        num_cores=NC, num_subcores=NS)
    body = functools.partial(_secgelu_body, n_chunks=n_chunks)
    return pl.kernel(
        body,
        out_type=jax.ShapeDtypeStruct(x.shape, jnp.float32),
        mesh=mesh,
        scratch_types=[
            pltpu.VMEM((TABLE_N,), jnp.float32),
            pltpu.VMEM((2 * TABLE_N,), jnp.float32),
            pltpu.VMEM((ROWS, COLS), jnp.float32),
            pltpu.VMEM((ROWS, COLS), jnp.float32),
            pltpu.VMEM((ROWS, COLS), jnp.float32),
            pltpu.VMEM((ROWS, COLS), jnp.float32),
            pltpu.SemaphoreType.DMA((NB,)),
            pltpu.SemaphoreType.DMA((NB,)),
        ],
        compiler_params=pltpu.CompilerParams(needs_layout_passes=False),
        name="secgelu_sc",
    )(x, table)
